# baseline (speedup 1.0000x reference)
import jax
import jax.numpy as jnp
from jax.experimental import pallas as pl
from jax.experimental.pallas import tpu as pltpu

S, B, D = 2048, 4, 2048
BS = 256
N = S // BS


def _concat_embed_body(t_ref, x_ref, table_ref, out_ref, emb_ref, gat_sems):
    i = pl.program_id(0)

    def gathers():
        return [
            pltpu.make_async_copy(
                table_ref.at[t_ref[b]], emb_ref.at[b], gat_sems.at[b])
            for b in range(B)
        ]

    @pl.when(i == 0)
    def _start_gather():
        for g in gathers():
            g.start()

    @pl.when(i < N)
    def _copy():
        out_ref[...] = x_ref[...]

    @pl.when(i == N)
    def _emit_embedding():
        for g in gathers():
            g.wait()
        out_ref[0] = emb_ref[...]


def kernel(x, t, table):
    return pl.pallas_call(
        _concat_embed_body,
        grid=(N + 1,),
        out_shape=jax.ShapeDtypeStruct((S + 1, B, D), x.dtype),
        in_specs=[
            pl.BlockSpec(memory_space=pltpu.SMEM),
            pl.BlockSpec((BS, B, D), lambda i: (jnp.minimum(i, N - 1), 0, 0)),
            pl.BlockSpec(memory_space=pl.ANY),
        ],
        out_specs=pl.BlockSpec((BS, B, D), lambda i: (i, 0, 0)),
        scratch_shapes=[
            pltpu.VMEM((B, D), jnp.float32),
            pltpu.SemaphoreType.DMA((B,)),
        ],
    )(t, x, table)
